# baseline (speedup 1.0000x reference)
import functools

import jax
import jax.numpy as jnp
import numpy as np
from jax.experimental import pallas as pl
from jax.experimental.pallas import tpu as pltpu

_NCODONS = 2
_N_STATES = 24
_N_PARAMS = 10


def _static_structure(nCodons=_NCODONS):
    offset = 8 + 3 * nCodons
    idx = [[0, 0], [0, 1], [1, 2], [2, 3]]
    idx += [[3 + i * 3, 4 + i * 3] for i in range(nCodons)]
    idx += [[4 + i * 3, 5 + i * 3] for i in range(nCodons)]
    idx += [[5 + i * 3, 6 + i * 3] for i in range(nCodons)]
    idx += [[3 + i * 3, offset + i * 3] for i in range(nCodons + 1)]
    idx += [[3 + nCodons * 3, 4 + nCodons * 3]]
    idx += [[offset + i * 3, offset + 1 + i * 3] for i in range(nCodons + 1)]
    idx += [[offset + 1 + i * 3, offset + 2 + i * 3] for i in range(nCodons + 1)]
    idx += [[offset + 2 + i * 3, 4 + i * 3] for i in range(nCodons + 1)]
    idx += [[offset + 2 + i * 3, offset + i * 3] for i in range(nCodons + 1)]
    i_del = [3 + i * 3 for i in range(nCodons) for j in range(nCodons - i)]
    j_del = [4 + j * 3 for i in range(1, nCodons + 1) for j in range(i, nCodons + 1)]
    idx += [[i, j] for i, j in zip(i_del, j_del)]
    idx += [[4 + nCodons * 3, 5 + nCodons * 3]]
    idx += [[5 + nCodons * 3, 6 + nCodons * 3]]
    idx += [[6 + nCodons * 3, 7 + nCodons * 3]]
    idx += [[7 + nCodons * 3, 7 + nCodons * 3]]
    idx += [[7 + nCodons * 3, 8 + nCodons * 3 + (nCodons + 1) * 3]]
    idx += [[8 + nCodons * 3 + (nCodons + 1) * 3,
             8 + nCodons * 3 + (nCodons + 1) * 3]]
    idx = np.array(idx, dtype=np.int32)

    nc = nCodons
    a, b, p, e = [], [], [], []

    def add(ai, bi, pi, ei):
        a.append(ai); b.append(bi); p.append(pi); e.append(ei)

    add(1.0, -1.0, 0, 1)
    add(0.0, 1.0, 0, 1)
    for _ in range(2):
        add(1.0, 0.0, 0, 1)
    k = 1
    for i in range(nc):
        add(0.0, 1.0, k + i, 1)
    k += nc
    for _ in range(2 * nc):
        add(1.0, 0.0, 0, 1)
    for i in range(nc + 1):
        add(0.0, 1.0, k + i, 1)
    k += nc + 1
    add(1.0, -1.0, k - 1, 1)
    for _ in range(2 * (nc + 1)):
        add(1.0, 0.0, 0, 1)
    for i in range(nc + 1):
        add(0.0, 1.0, k + i, 1)
    for i in range(nc + 1):
        add(1.0, -1.0, k + i, 1)
    k += nc + 1
    for i, j in zip(i_del, j_del):
        add(1.0, -1.0, k, 1 + int((j - i) / 3))
    k += 1
    for _ in range(6):
        add(1.0, 0.0, 0, 1)

    K = len(idx)
    assert len(a) == K
    coef_a = np.asarray(a, np.float32)
    coef_b = np.asarray(b, np.float32)
    par = np.asarray(p, np.int32)
    exps = np.asarray(e, np.float32)

    gt = np.zeros((_N_PARAMS, K), np.float32)
    gt[par, np.arange(K)] = 1.0
    rt = np.zeros((_N_STATES, K), np.float32)
    rt[idx[:, 0], np.arange(K)] = 1.0
    cm = np.zeros((K, _N_STATES), np.float32)
    cm[np.arange(K), idx[:, 1]] = 1.0
    return coef_a, coef_b, exps, gt, rt, cm


_COEF_A, _COEF_B, _EXPS, _GT, _RT, _CM = _static_structure()
_K = _GT.shape[1]
_M2 = (_EXPS >= 2.0).astype(np.float32)[None, :]
_M3 = (_EXPS >= 3.0).astype(np.float32)[None, :]


def _fused_body(w_ref, gt_ref, ca_ref, cb_ref, m2_ref, m3_ref, rt_ref,
                cm_ref, alpha_ref, out_ref):
    w = w_ref[...]
    gw = jnp.dot(w, gt_ref[...], preferred_element_type=jnp.float32)
    e2 = 1.0 + m2_ref[...] * (gw - 1.0)
    e3 = 1.0 + m3_ref[...] * (gw - 1.0)
    vals = ca_ref[...] + cb_ref[...] * gw * e2 * e3
    evals = jnp.exp(vals)
    emat = jnp.dot(rt_ref[...] * evals, cm_ref[...],
                   preferred_element_type=jnp.float32)
    inv = 1.0 / jnp.sum(emat, axis=1, keepdims=True)
    a_mat = emat * inv
    out_ref[...] = jnp.dot(alpha_ref[...], a_mat,
                           preferred_element_type=jnp.float32)


def _const_spec(shape):
    return pl.BlockSpec(shape, lambda i: tuple(0 for _ in shape))


@jax.jit
def kernel(alpha, transition_kernel):
    n = alpha.shape[0]
    blk = 8192
    grid = n // blk
    w2 = transition_kernel.reshape(1, _N_PARAMS)
    consts = (jnp.asarray(_GT), jnp.asarray(_COEF_A[None, :]),
              jnp.asarray(_COEF_B[None, :]), jnp.asarray(_M2),
              jnp.asarray(_M3), jnp.asarray(_RT), jnp.asarray(_CM))
    return pl.pallas_call(
        _fused_body,
        grid=(grid,),
        in_specs=[
            _const_spec((1, _N_PARAMS)),
            _const_spec((_N_PARAMS, _K)),
            _const_spec((1, _K)),
            _const_spec((1, _K)),
            _const_spec((1, _K)),
            _const_spec((1, _K)),
            _const_spec((_N_STATES, _K)),
            _const_spec((_K, _N_STATES)),
            pl.BlockSpec((blk, _N_STATES), lambda i: (i, 0)),
        ],
        out_specs=pl.BlockSpec((blk, _N_STATES), lambda i: (i, 0)),
        out_shape=jax.ShapeDtypeStruct((n, _N_STATES), jnp.float32),
        compiler_params=pltpu.CompilerParams(
            dimension_semantics=("arbitrary",),
        ),
    )(w2, *consts, alpha)
